# Initial kernel scaffold; baseline (speedup 1.0000x reference)
import jax
import jax.numpy as jnp
from jax.experimental import pallas as pl


def kernel(x):
    raise NotImplementedError("write your pallas kernel here")


# speedup vs baseline: 1922.4320x; 1922.4320x over previous
import functools

import jax
import jax.numpy as jnp
from jax import lax
from jax.experimental import pallas as pl
from jax.experimental.pallas import tpu as pltpu
from jax.experimental.pallas import tpu_sc as plsc

_VOCAB_SIZE = 1000

_NC, _NS, _L = 2, 16, 16
_NW = _NC * _NS


@functools.cache
def _build(n):
    assert n % (_NW * _L) == 0
    per_w = n // _NW
    mesh = plsc.VectorSubcoreMesh(core_axis_name="c", subcore_axis_name="s")

    @functools.partial(
        pl.kernel,
        out_type=jax.ShapeDtypeStruct((n,), jnp.int32),
        mesh=mesh,
        scratch_types=[
            pltpu.VMEM((per_w,), jnp.int32),
            pltpu.VMEM((per_w,), jnp.int32),
        ],
    )
    def body(x_hbm, out_hbm, in_v, out_v):
        wid = lax.axis_index("s") * _NC + lax.axis_index("c")
        base = wid * per_w
        pltpu.sync_copy(x_hbm.at[pl.ds(base, per_w)], in_v)

        def step(i, carry):
            v = in_v[pl.ds(i * _L, _L)]
            ok = (v >= 0) & (v < _VOCAB_SIZE)
            out_v[pl.ds(i * _L, _L)] = jnp.where(ok, v, jnp.int32(-1))
            return carry

        lax.fori_loop(0, per_w // _L, step, 0)
        pltpu.sync_copy(out_v, out_hbm.at[pl.ds(base, per_w)])

    return body


def kernel(x):
    flat = x.reshape(-1)
    out = _build(flat.shape[0])(flat)
    return out.reshape(x.shape)


# speedup vs baseline: 3578.1799x; 1.8613x over previous
import functools

import jax
import jax.numpy as jnp
from jax import lax
from jax.experimental import pallas as pl
from jax.experimental.pallas import tpu as pltpu
from jax.experimental.pallas import tpu_sc as plsc

_VOCAB_SIZE = 1000

_NC, _NS, _L = 2, 16, 16
_NW = _NC * _NS


@functools.cache
def _build(nrows, ncols):
    assert nrows % _NW == 0
    rows_w = nrows // _NW
    col_offs = list(range(0, ncols - _L + 1, _L))
    if col_offs[-1] != ncols - _L:
        col_offs.append(ncols - _L)

    mesh = plsc.VectorSubcoreMesh(core_axis_name="c", subcore_axis_name="s")

    @functools.partial(
        pl.kernel,
        out_type=jax.ShapeDtypeStruct((nrows, ncols), jnp.int32),
        mesh=mesh,
        scratch_types=[pltpu.VMEM((rows_w, ncols), jnp.int32)],
    )
    def body(x_hbm, out_hbm, buf):
        wid = lax.axis_index("s") * _NC + lax.axis_index("c")
        base = wid * rows_w
        pltpu.sync_copy(x_hbm.at[pl.ds(base, rows_w)], buf)

        def step(r, carry):
            for c in col_offs:
                v = buf[r, pl.ds(c, _L)]
                ok = (v >= 0) & (v < _VOCAB_SIZE)
                buf[r, pl.ds(c, _L)] = jnp.where(ok, v, jnp.int32(-1))
            return carry

        lax.fori_loop(0, rows_w, step, 0)
        pltpu.sync_copy(buf, out_hbm.at[pl.ds(base, rows_w)])

    return body


def kernel(x):
    return _build(*x.shape)(x)


# speedup vs baseline: 3590.5356x; 1.0035x over previous
import functools

import jax
import jax.numpy as jnp
from jax import lax
from jax.experimental import pallas as pl
from jax.experimental.pallas import tpu as pltpu
from jax.experimental.pallas import tpu_sc as plsc

_VOCAB_SIZE = 1000

_NC, _NS, _L = 2, 16, 16
_NW = _NC * _NS


@functools.cache
def _build(nrows, ncols):
    assert nrows % _NW == 0
    rows_w = nrows // _NW
    col_offs = list(range(0, ncols - _L + 1, _L))
    if col_offs[-1] != ncols - _L:
        col_offs.append(ncols - _L)

    mesh = plsc.VectorSubcoreMesh(core_axis_name="c", subcore_axis_name="s")

    @functools.partial(
        pl.kernel,
        out_type=jax.ShapeDtypeStruct((nrows, ncols), jnp.int32),
        mesh=mesh,
        scratch_types=[pltpu.VMEM((rows_w, ncols), jnp.int32)],
        compiler_params=pltpu.CompilerParams(use_tc_tiling_on_sc=True),
    )
    def body(x_hbm, out_hbm, buf):
        wid = lax.axis_index("s") * _NC + lax.axis_index("c")
        base = wid * rows_w
        pltpu.sync_copy(x_hbm.at[pl.ds(base, rows_w)], buf)

        def step(r, carry):
            for c in col_offs:
                v = buf[r, pl.ds(c, _L)]
                ok = (v >= 0) & (v < _VOCAB_SIZE)
                buf[r, pl.ds(c, _L)] = jnp.where(ok, v, jnp.int32(-1))
            return carry

        lax.fori_loop(0, rows_w, step, 0)
        pltpu.sync_copy(buf, out_hbm.at[pl.ds(base, rows_w)])

    return body


def kernel(x):
    return _build(*x.shape)(x)


# speedup vs baseline: 6156.7148x; 1.7147x over previous
import functools

import jax
import jax.numpy as jnp
from jax import lax
from jax.experimental import pallas as pl
from jax.experimental.pallas import tpu as pltpu
from jax.experimental.pallas import tpu_sc as plsc

_VOCAB_SIZE = 1000

_NC, _NS, _L = 2, 16, 16
_NW = _NC * _NS


@functools.cache
def _build(nrows, ncols):
    assert ncols % (_NW * _L) == 0
    cols_w = ncols // _NW

    mesh = plsc.VectorSubcoreMesh(core_axis_name="c", subcore_axis_name="s")

    @functools.partial(
        pl.kernel,
        out_type=jax.ShapeDtypeStruct((nrows, ncols), jnp.int32),
        mesh=mesh,
        scratch_types=[pltpu.VMEM((nrows, cols_w), jnp.int32)],
        compiler_params=pltpu.CompilerParams(use_tc_tiling_on_sc=True),
    )
    def body(x_hbm, out_hbm, buf):
        wid = lax.axis_index("s") * _NC + lax.axis_index("c")
        base = wid * cols_w
        pltpu.sync_copy(x_hbm.at[:, pl.ds(base, cols_w)], buf)

        def step(r, carry):
            for ci in range(cols_w // _L):
                v = buf[r, pl.ds(ci * _L, _L)]
                ok = plsc.bitcast(v, jnp.uint32) < _VOCAB_SIZE
                buf[r, pl.ds(ci * _L, _L)] = jnp.where(ok, v, jnp.int32(-1))
            return carry

        lax.fori_loop(0, nrows, step, 0)
        pltpu.sync_copy(buf, out_hbm.at[:, pl.ds(base, cols_w)])

    return body


def kernel(x):
    xt = x.T
    return _build(*xt.shape)(xt).T


# speedup vs baseline: 6177.1304x; 1.0033x over previous
import functools

import jax
import jax.numpy as jnp
from jax import lax
from jax.experimental import pallas as pl
from jax.experimental.pallas import tpu as pltpu
from jax.experimental.pallas import tpu_sc as plsc

_VOCAB_SIZE = 1000

_NC, _NS, _L = 2, 16, 16
_NW = _NC * _NS


@functools.cache
def _build(nrows, ncols):
    assert ncols % (_NW * _L) == 0
    cols_w = ncols // _NW

    mesh = plsc.VectorSubcoreMesh(core_axis_name="c", subcore_axis_name="s")

    @functools.partial(
        pl.kernel,
        out_type=jax.ShapeDtypeStruct((nrows, ncols), jnp.int32),
        mesh=mesh,
        scratch_types=[pltpu.VMEM((nrows, cols_w), jnp.int32)],
        compiler_params=pltpu.CompilerParams(
            use_tc_tiling_on_sc=True, skip_device_barrier=True
        ),
    )
    def body(x_hbm, out_hbm, buf):
        wid = lax.axis_index("s") * _NC + lax.axis_index("c")
        base = wid * cols_w
        pltpu.sync_copy(x_hbm.at[:, pl.ds(base, cols_w)], buf)

        def step(r, carry):
            for ci in range(cols_w // _L):
                v = buf[r, pl.ds(ci * _L, _L)]
                ok = plsc.bitcast(v, jnp.uint32) < _VOCAB_SIZE
                buf[r, pl.ds(ci * _L, _L)] = jnp.where(ok, v, jnp.int32(-1))
            return carry

        lax.fori_loop(0, nrows, step, 0)
        pltpu.sync_copy(buf, out_hbm.at[:, pl.ds(base, cols_w)])

    return body


def kernel(x):
    xt = x.T
    return _build(*xt.shape)(xt).T


# speedup vs baseline: 6481.8693x; 1.0493x over previous
import functools

import jax
import jax.numpy as jnp
from jax import lax
from jax.experimental import pallas as pl
from jax.experimental.pallas import tpu as pltpu
from jax.experimental.pallas import tpu_sc as plsc

_VOCAB_SIZE = 1000

_NC, _NS, _L = 2, 16, 16
_NW = _NC * _NS
_CH = 4


@functools.cache
def _build(nrows, ncols):
    assert ncols % (_NW * _CH * 128) == 0
    cols_w = ncols // _NW
    cw = cols_w // _CH
    vecs = cw // _L
    assert nrows % 4 == 0 or nrows % 2 == 0 or True

    mesh = plsc.VectorSubcoreMesh(core_axis_name="c", subcore_axis_name="s")

    @functools.partial(
        pl.kernel,
        out_type=jax.ShapeDtypeStruct((nrows, ncols), jnp.int32),
        mesh=mesh,
        scratch_types=(
            [pltpu.VMEM((nrows, cw), jnp.int32) for _ in range(_CH)]
            + [pltpu.SemaphoreType.DMA for _ in range(2 * _CH)]
        ),
        compiler_params=pltpu.CompilerParams(use_tc_tiling_on_sc=True),
    )
    def body(x_hbm, out_hbm, *scratch):
        bufs = scratch[:_CH]
        sin = scratch[_CH : 2 * _CH]
        sout = scratch[2 * _CH :]
        wid = lax.axis_index("s") * _NC + lax.axis_index("c")
        base = wid * cols_w

        h_in = [
            pltpu.async_copy(
                x_hbm.at[:, pl.ds(base + c * cw, cw)], bufs[c], sin[c]
            )
            for c in range(_CH)
        ]
        h_out = []
        for c in range(_CH):
            h_in[c].wait()
            buf = bufs[c]

            def step(r, carry, buf=buf):
                for ci in range(vecs):
                    v = buf[r, pl.ds(ci * _L, _L)]
                    ok = plsc.bitcast(v, jnp.uint32) < _VOCAB_SIZE
                    buf[r, pl.ds(ci * _L, _L)] = jnp.where(ok, v, jnp.int32(-1))
                return carry

            lax.fori_loop(0, nrows, step, 0)
            h_out.append(
                pltpu.async_copy(
                    buf, out_hbm.at[:, pl.ds(base + c * cw, cw)], sout[c]
                )
            )
        for h in h_out:
            h.wait()

    return body


def kernel(x):
    xt = x.T
    return _build(*xt.shape)(xt).T


# speedup vs baseline: 6491.7876x; 1.0015x over previous
import functools

import jax
import jax.numpy as jnp
from jax import lax
from jax.experimental import pallas as pl
from jax.experimental.pallas import tpu as pltpu
from jax.experimental.pallas import tpu_sc as plsc

_VOCAB_SIZE = 1000

_NC, _NS, _L = 2, 16, 16
_NW = _NC * _NS
_CH = 4


@functools.cache
def _build(nrows, ncols):
    assert ncols % (_NW * _CH * 128) == 0
    cols_w = ncols // _NW
    cw = cols_w // _CH
    vecs = cw // _L
    assert nrows % 4 == 0 or nrows % 2 == 0 or True

    mesh = plsc.VectorSubcoreMesh(core_axis_name="c", subcore_axis_name="s")

    @functools.partial(
        pl.kernel,
        out_type=jax.ShapeDtypeStruct((nrows, ncols), jnp.int32),
        mesh=mesh,
        scratch_types=(
            [pltpu.VMEM((nrows, cw), jnp.int32) for _ in range(_CH)]
            + [pltpu.SemaphoreType.DMA for _ in range(2 * _CH)]
        ),
        compiler_params=pltpu.CompilerParams(use_tc_tiling_on_sc=True),
    )
    def body(x_hbm, out_hbm, *scratch):
        bufs = scratch[:_CH]
        sin = scratch[_CH : 2 * _CH]
        sout = scratch[2 * _CH :]
        wid = lax.axis_index("s") * _NC + lax.axis_index("c")
        base = wid * cols_w

        h_in = [
            pltpu.async_copy(
                x_hbm.at[:, pl.ds(base + c * cw, cw)], bufs[c], sin[c]
            )
            for c in range(_CH)
        ]
        h_out = []
        for c in range(_CH):
            h_in[c].wait()
            buf = bufs[c]

            @plsc.parallel_loop(0, nrows, unroll=2)
            def _(r, buf=buf):
                for ci in range(vecs):
                    v = buf[r, pl.ds(ci * _L, _L)]
                    ok = plsc.bitcast(v, jnp.uint32) < _VOCAB_SIZE
                    buf[r, pl.ds(ci * _L, _L)] = jnp.where(ok, v, jnp.int32(-1))
            h_out.append(
                pltpu.async_copy(
                    buf, out_hbm.at[:, pl.ds(base + c * cw, cw)], sout[c]
                )
            )
        for h in h_out:
            h.wait()

    return body


def kernel(x):
    xt = x.T
    return _build(*xt.shape)(xt).T
